# baseline (speedup 1.0000x reference)
import functools
import math

import jax
import jax.numpy as jnp
from jax import lax
from jax.experimental import pallas as pl
from jax.experimental.pallas import tpu as pltpu
from jax.experimental.pallas import tpu_sc as plsc

_B = 1024
_K = 128
_N = 100000
_BETA = 0.05
_KNN = 6
_GRP = 128
_CHUNK = _GRP * 128
_NFULL = _N // _CHUNK
_TAIL = _N - _NFULL * _CHUNK
_TGRP = -(-_TAIL // 128)
_TPART = _TAIL - (_TGRP - 1) * 128
_NEG = -1e30
_LN2 = math.log(2.0)


def _sc_gather(em, idx):
    info = plsc.get_sparse_core_info()
    nw = info.num_cores * info.num_subcores
    bpw = _B // nw
    mesh = plsc.VectorSubcoreMesh(core_axis_name="c", subcore_axis_name="s")

    @functools.partial(
        pl.kernel, mesh=mesh,
        out_type=jax.ShapeDtypeStruct((_B, _K), jnp.float32),
        scratch_types=[
            pltpu.VMEM((bpw,), jnp.int32),
            pltpu.VMEM((bpw, _K), jnp.float32),
            pltpu.SemaphoreType.DMA,
        ],
    )
    def gather_k(em_hbm, idx_hbm, out_hbm, idx_v, rows_v, sem):
        wid = lax.axis_index("s") * info.num_cores + lax.axis_index("c")
        base = wid * bpw
        pltpu.sync_copy(idx_hbm.at[pl.ds(base, bpw)], idx_v)
        pltpu.async_copy(em_hbm.at[idx_v], rows_v, sem).wait()
        pltpu.sync_copy(rows_v, out_hbm.at[pl.ds(base, bpw)])

    return gather_k(em, idx)


def _cmp(a, b):
    return jnp.maximum(a, b), jnp.minimum(a, b)


def _merge22(a, b):
    o1, t1 = _cmp(a[0], b[0])
    t2, o4 = _cmp(a[1], b[1])
    o2, o3 = _cmp(t1, t2)
    return [o1, o2, o3, o4]


def _merge44(a, b):
    p = _merge22([a[0], a[2]], [b[0], b[2]])
    q = _merge22([a[1], a[3]], [b[1], b[3]])
    out = [p[0]]
    for i in range(3):
        hi, lo = _cmp(p[i + 1], q[i])
        out += [hi, lo]
    out.append(q[3])
    return out


def _merge_topk(a, b, k):
    out = []
    for i in range(1, k + 1):
        cands = []
        for j in range(0, i + 1):
            ai = i - j
            if ai > len(a) or j > len(b):
                continue
            if ai == 0:
                cands.append(b[j - 1])
            elif j == 0:
                cands.append(a[ai - 1])
            else:
                cands.append(jnp.minimum(a[ai - 1], b[j - 1]))
        m = cands[0]
        for c in cands[1:]:
            m = jnp.maximum(m, c)
        out.append(m)
    return out


def _top6_of_16(xs):
    pairs = [_cmp(xs[2 * i], xs[2 * i + 1]) for i in range(8)]
    quads = [_merge22(list(pairs[2 * i]), list(pairs[2 * i + 1]))
             for i in range(4)]
    o8a = _merge44(quads[0], quads[1])
    o8b = _merge44(quads[2], quads[3])
    return _merge_topk(o8a, o8b, _KNN)


def _top6_net(xs):
    if len(xs) <= 16:
        xs = xs + [None] * (16 - len(xs))
        xs = [x if x is not None else jnp.full_like(xs[0], _NEG) for x in xs]
        return _top6_of_16(xs)
    a = _top6_net(xs[:16])
    b = _top6_net(xs[16:])
    return _merge_topk(a, b, _KNN)


def _fused_body(f_ref, em_ref, labrow_ref, out_ref, top_ref, s_ref):
    j = pl.program_id(0)

    @pl.when(j == 0)
    def _init():
        top_ref[...] = jnp.full((_B, _KNN * 128), _NEG, jnp.float32)
        s_ref[...] = jnp.zeros((_B, 128), jnp.float32)

    f = f_ref[...]

    def dot_group(src_ref, g):
        return jax.lax.dot_general(
            f, src_ref[g * 128:(g + 1) * 128, :], (((1,), (1,)), ((), ())),
            preferred_element_type=jnp.float32)

    def process(make_x, ngroups):
        xs = [make_x(g) for g in range(ngroups)]
        new6 = _top6_net(xs)

        tops = [top_ref[:, i * 128:(i + 1) * 128] for i in range(_KNN)]
        m_old = tops[0]
        tops = _merge_topk(tops, new6, _KNN)
        for i in range(_KNN):
            top_ref[:, i * 128:(i + 1) * 128] = tops[i]
        m_new = tops[0]

        s = s_ref[...] * jnp.exp2(m_old - m_new)
        for g in range(ngroups):
            s = s + jnp.exp2(make_x(g) - m_new)
        s_ref[...] = s
        return m_new, s

    @pl.when(j < _NFULL)
    def _main():
        process(lambda g: dot_group(em_ref, g), _GRP)

    @pl.when(j == _NFULL)
    def _tail_and_finish():
        lane = jax.lax.broadcasted_iota(jnp.int32, (_B, 128), 1)

        def make_x(g):
            x = dot_group(em_ref, g)
            if g == _TGRP - 1:
                x = jnp.where(lane < _TPART, x, jnp.float32(_NEG))
            return x

        m_new, s = process(make_x, _TGRP)

        lab = jnp.sum(f * labrow_ref[...], axis=1, keepdims=True)
        mrow = jnp.max(m_new, axis=1, keepdims=True)
        srow = jnp.sum(s * jnp.exp2(m_new - mrow), axis=1, keepdims=True)
        lse = mrow + jnp.log2(srow)
        merged = top_ref[...]
        sum_top = jnp.zeros((_B, 1), jnp.float32)
        mk = jnp.zeros((_B, 1), jnp.float32)
        for _ in range(_KNN):
            mk = jnp.max(merged, axis=1, keepdims=True)
            sum_top = sum_top + mk
            merged = jnp.where(merged == mk, _NEG, merged)
        t = (lab >= mk).astype(jnp.float32)
        loss = (15.0 - 2.0 * t) * lse - 2.0 * sum_top - (3.0 - 2.0 * t) * lab
        out_ref[...] = jnp.sum(loss, keepdims=True)[:, :1] * (_LN2 / _B)


@jax.jit
def _run(tgt_feature, tgt_label, em):
    f_scaled = tgt_feature * (1.0 / (_BETA * _LN2))
    labrows = _sc_gather(em, tgt_label.astype(jnp.int32))
    out = pl.pallas_call(
        _fused_body,
        grid=(_NFULL + 1,),
        in_specs=[
            pl.BlockSpec((_B, _K), lambda j: (0, 0)),
            pl.BlockSpec((_CHUNK, _K), lambda j: (j, 0)),
            pl.BlockSpec((_B, _K), lambda j: (0, 0)),
        ],
        out_specs=pl.BlockSpec((1, 1), lambda j: (0, 0)),
        out_shape=jax.ShapeDtypeStruct((1, 1), jnp.float32),
        scratch_shapes=[
            pltpu.VMEM((_B, _KNN * 128), jnp.float32),
            pltpu.VMEM((_B, 128), jnp.float32),
        ],
        compiler_params=pltpu.CompilerParams(
            dimension_semantics=("arbitrary",)),
    )(f_scaled, em, labrows)
    return out[0, 0]


def kernel(tgt_feature, tgt_label, epoch, em):
    del epoch
    return _run(tgt_feature, tgt_label, em)


# speedup vs baseline: 1.3251x; 1.3251x over previous
import functools
import math

import jax
import jax.numpy as jnp
from jax import lax
from jax.experimental import pallas as pl
from jax.experimental.pallas import tpu as pltpu
from jax.experimental.pallas import tpu_sc as plsc

_B = 1024
_K = 128
_N = 100000
_BETA = 0.05
_KNN = 6
_GRP = 64
_CHUNK = _GRP * 128
_NFULL = _N // _CHUNK
_TAIL = _N - _NFULL * _CHUNK
_TGRP = -(-_TAIL // 128)
_TPART = _TAIL - (_TGRP - 1) * 128
_NEG = -1e30
_LN2 = math.log(2.0)


def _sc_gather(em, idx):
    info = plsc.get_sparse_core_info()
    nw = info.num_cores * info.num_subcores
    bpw = _B // nw
    mesh = plsc.VectorSubcoreMesh(core_axis_name="c", subcore_axis_name="s")

    @functools.partial(
        pl.kernel, mesh=mesh,
        out_type=jax.ShapeDtypeStruct((_B, _K), jnp.float32),
        scratch_types=[
            pltpu.VMEM((bpw,), jnp.int32),
            pltpu.VMEM((bpw, _K), jnp.float32),
            pltpu.SemaphoreType.DMA,
        ],
    )
    def gather_k(em_hbm, idx_hbm, out_hbm, idx_v, rows_v, sem):
        wid = lax.axis_index("s") * info.num_cores + lax.axis_index("c")
        base = wid * bpw
        pltpu.sync_copy(idx_hbm.at[pl.ds(base, bpw)], idx_v)
        pltpu.async_copy(em_hbm.at[idx_v], rows_v, sem).wait()
        pltpu.sync_copy(rows_v, out_hbm.at[pl.ds(base, bpw)])

    return gather_k(em, idx)


def _cmp(a, b):
    return jnp.maximum(a, b), jnp.minimum(a, b)


def _merge22(a, b):
    o1, t1 = _cmp(a[0], b[0])
    t2, o4 = _cmp(a[1], b[1])
    o2, o3 = _cmp(t1, t2)
    return [o1, o2, o3, o4]


def _merge44(a, b):
    p = _merge22([a[0], a[2]], [b[0], b[2]])
    q = _merge22([a[1], a[3]], [b[1], b[3]])
    out = [p[0]]
    for i in range(3):
        hi, lo = _cmp(p[i + 1], q[i])
        out += [hi, lo]
    out.append(q[3])
    return out


def _merge_topk(a, b, k):
    out = []
    for i in range(1, k + 1):
        cands = []
        for j in range(0, i + 1):
            ai = i - j
            if ai > len(a) or j > len(b):
                continue
            if ai == 0:
                cands.append(b[j - 1])
            elif j == 0:
                cands.append(a[ai - 1])
            else:
                cands.append(jnp.minimum(a[ai - 1], b[j - 1]))
        m = cands[0]
        for c in cands[1:]:
            m = jnp.maximum(m, c)
        out.append(m)
    return out


def _top6_of_16(xs):
    pairs = [_cmp(xs[2 * i], xs[2 * i + 1]) for i in range(8)]
    quads = [_merge22(list(pairs[2 * i]), list(pairs[2 * i + 1]))
             for i in range(4)]
    o8a = _merge44(quads[0], quads[1])
    o8b = _merge44(quads[2], quads[3])
    return _merge_topk(o8a, o8b, _KNN)


def _top6_net(xs):
    if len(xs) <= 16:
        xs = xs + [None] * (16 - len(xs))
        xs = [x if x is not None else jnp.full_like(xs[0], _NEG) for x in xs]
        return _top6_of_16(xs)
    a = _top6_net(xs[:16])
    b = _top6_net(xs[16:])
    return _merge_topk(a, b, _KNN)


def _fused_body(f_ref, em_ref, labrow_ref, out_ref, top_ref, s_ref):
    j = pl.program_id(0)

    @pl.when(j == 0)
    def _init():
        top_ref[...] = jnp.full((_B, _KNN * 128), _NEG, jnp.float32)
        s_ref[...] = jnp.zeros((_B, 128), jnp.float32)

    f = f_ref[...]

    def dot_group(src_ref, g):
        return jax.lax.dot_general(
            f, src_ref[g * 128:(g + 1) * 128, :], (((1,), (1,)), ((), ())),
            preferred_element_type=jnp.float32)

    def process(make_x, ngroups):
        xs = [make_x(g) for g in range(ngroups)]
        new6 = _top6_net(xs)

        tops = [top_ref[:, i * 128:(i + 1) * 128] for i in range(_KNN)]
        m_old = tops[0]
        tops = _merge_topk(tops, new6, _KNN)
        for i in range(_KNN):
            top_ref[:, i * 128:(i + 1) * 128] = tops[i]
        m_new = tops[0]

        s = s_ref[...] * jnp.exp2(m_old - m_new)
        for g in range(ngroups):
            s = s + jnp.exp2(make_x(g) - m_new)
        s_ref[...] = s
        return m_new, s

    @pl.when(j < _NFULL)
    def _main():
        process(lambda g: dot_group(em_ref, g), _GRP)

    @pl.when(j == _NFULL)
    def _tail_and_finish():
        lane = jax.lax.broadcasted_iota(jnp.int32, (_B, 128), 1)

        def make_x(g):
            x = dot_group(em_ref, g)
            if g == _TGRP - 1:
                x = jnp.where(lane < _TPART, x, jnp.float32(_NEG))
            return x

        m_new, s = process(make_x, _TGRP)

        lab = jnp.sum(f * labrow_ref[...], axis=1, keepdims=True)
        mrow = jnp.max(m_new, axis=1, keepdims=True)
        srow = jnp.sum(s * jnp.exp2(m_new - mrow), axis=1, keepdims=True)
        lse = mrow + jnp.log2(srow)
        merged = top_ref[...]
        sum_top = jnp.zeros((_B, 1), jnp.float32)
        mk = jnp.zeros((_B, 1), jnp.float32)
        for _ in range(_KNN):
            mk = jnp.max(merged, axis=1, keepdims=True)
            sum_top = sum_top + mk
            merged = jnp.where(merged == mk, _NEG, merged)
        t = (lab >= mk).astype(jnp.float32)
        loss = (15.0 - 2.0 * t) * lse - 2.0 * sum_top - (3.0 - 2.0 * t) * lab
        out_ref[...] = jnp.sum(loss, keepdims=True)[:, :1] * (_LN2 / _B)


@jax.jit
def _run(tgt_feature, tgt_label, em):
    f_scaled = tgt_feature * (1.0 / (_BETA * _LN2))
    labrows = _sc_gather(em, tgt_label.astype(jnp.int32))
    out = pl.pallas_call(
        _fused_body,
        grid=(_NFULL + 1,),
        in_specs=[
            pl.BlockSpec((_B, _K), lambda j: (0, 0)),
            pl.BlockSpec((_CHUNK, _K), lambda j: (j, 0)),
            pl.BlockSpec((_B, _K), lambda j: (0, 0)),
        ],
        out_specs=pl.BlockSpec((1, 1), lambda j: (0, 0)),
        out_shape=jax.ShapeDtypeStruct((1, 1), jnp.float32),
        scratch_shapes=[
            pltpu.VMEM((_B, _KNN * 128), jnp.float32),
            pltpu.VMEM((_B, 128), jnp.float32),
        ],
        compiler_params=pltpu.CompilerParams(
            dimension_semantics=("arbitrary",)),
    )(f_scaled, em, labrows)
    return out[0, 0]


def kernel(tgt_feature, tgt_label, epoch, em):
    del epoch
    return _run(tgt_feature, tgt_label, em)
